# initial kernel scaffold (unmeasured)
import jax
import jax.numpy as jnp
from jax import lax
from jax.experimental import pallas as pl
from jax.experimental.pallas import tpu as pltpu

N_DEV = 16
M, N = 8192, 4096
MC = M // N_DEV

_PLANE = {(0, 0): 0, (1, 0): 1, (1, 1): 2, (0, 1): 3}
_RING_COORDS = (
    [(0, 0, z) for z in range(4)]
    + [(1, 0, z) for z in range(3, -1, -1)]
    + [(1, 1, z) for z in range(4)]
    + [(0, 1, z) for z in range(3, -1, -1)]
)
_RING_MESH = [4 * z + _PLANE[(x, y)] for (x, y, z) in _RING_COORDS]
_MESH_TO_RING = [0] * N_DEV
for _r, _p in enumerate(_RING_MESH):
    _MESH_TO_RING[_p] = _r
_RIGHT = [_RING_MESH[(_MESH_TO_RING[p] + 1) % N_DEV] for p in range(N_DEV)]


def _body(scal_ref, partial_ref, out_ref, acc_ref, loc_ref, rsbuf_ref,
          copy_sems, rs_send, rs_recv, ag_send, ag_recv):
    my_ring = scal_ref[0]
    right = scal_ref[1]

    def prow(ref, c):
        return ref.at[pl.ds(c * MC, MC), :]

    def chunk_id(k):
        return lax.rem(my_ring - k + 2 * N_DEV, N_DEV)

    cp = pltpu.make_async_copy(
        prow(partial_ref, chunk_id(0)), acc_ref.at[0], copy_sems.at[0])
    cp.start()
    cp.wait()

    for s in range(N_DEV - 1):
        cur, nxt = s % 2, (s + 1) % 2
        rdma = pltpu.make_async_remote_copy(
            src_ref=acc_ref.at[cur],
            dst_ref=rsbuf_ref.at[s],
            send_sem=rs_send.at[s],
            recv_sem=rs_recv.at[s],
            device_id=right,
            device_id_type=pl.DeviceIdType.LOGICAL,
        )
        rdma.start()
        cp_loc = pltpu.make_async_copy(
            prow(partial_ref, chunk_id(s + 1)), loc_ref, copy_sems.at[0])
        cp_loc.start()
        rdma.wait()
        cp_in = pltpu.make_async_copy(
            rsbuf_ref.at[s], acc_ref.at[nxt], copy_sems.at[1])
        cp_in.start()
        cp_loc.wait()
        cp_in.wait()
        acc_ref[nxt] = acc_ref[nxt] + loc_ref[...]

    fin = (N_DEV - 1) % 2
    acc_ref[fin] = jnp.maximum(acc_ref[fin], 0.0)
    c_fin = chunk_id(N_DEV - 1)
    cp_out = pltpu.make_async_copy(
        acc_ref.at[fin], prow(out_ref, c_fin), copy_sems.at[0])
    cp_out.start()
    cp_out.wait()

    for t in range(N_DEV - 1):
        g = lax.rem(my_ring + 1 - t + 2 * N_DEV, N_DEV)
        src = acc_ref.at[fin] if t == 0 else prow(out_ref, g)
        rdma = pltpu.make_async_remote_copy(
            src_ref=src,
            dst_ref=prow(out_ref, g),
            send_sem=ag_send.at[t],
            recv_sem=ag_recv.at[t],
            device_id=right,
            device_id_type=pl.DeviceIdType.LOGICAL,
        )
        rdma.start()
        rdma.wait()


def kernel(x, w_mat):
    partial = jnp.dot(x, w_mat, preferred_element_type=jnp.float32)

    p = lax.axis_index("i")
    my_ring = jnp.asarray(_MESH_TO_RING, jnp.int32)[p]
    right = jnp.asarray(_RIGHT, jnp.int32)[p]
    scal = jnp.stack([my_ring, right]).astype(jnp.int32)

    return pl.pallas_call(
        _body,
        out_shape=jax.ShapeDtypeStruct((M, N), jnp.float32),
        in_specs=[
            pl.BlockSpec(memory_space=pltpu.SMEM),
            pl.BlockSpec(memory_space=pltpu.ANY),
        ],
        out_specs=pl.BlockSpec(memory_space=pltpu.ANY),
        scratch_shapes=[
            pltpu.VMEM((2, MC, N), jnp.float32),
            pltpu.VMEM((MC, N), jnp.float32),
            pltpu.ANY((N_DEV - 1, MC, N), jnp.float32),
            pltpu.SemaphoreType.DMA((2,)),
            pltpu.SemaphoreType.DMA((N_DEV - 1,)),
            pltpu.SemaphoreType.DMA((N_DEV - 1,)),
            pltpu.SemaphoreType.DMA((N_DEV - 1,)),
            pltpu.SemaphoreType.DMA((N_DEV - 1,)),
        ],
        compiler_params=pltpu.CompilerParams(collective_id=0),
    )(scal, partial)


# baseline (device time: 2929550 ns/iter reference)
import jax
import jax.numpy as jnp
from jax import lax
from jax.experimental import pallas as pl
from jax.experimental.pallas import tpu as pltpu

N_DEV = 16
M, N = 8192, 4096
MC = M // N_DEV

_PLANE = {(0, 0): 0, (1, 0): 1, (1, 1): 2, (0, 1): 3}
_RING_COORDS = (
    [(0, 0, z) for z in range(4)]
    + [(1, 0, z) for z in range(3, -1, -1)]
    + [(1, 1, z) for z in range(4)]
    + [(0, 1, z) for z in range(3, -1, -1)]
)
_RING_MESH = [4 * z + _PLANE[(x, y)] for (x, y, z) in _RING_COORDS]
_MESH_TO_RING = [0] * N_DEV
for _r, _p in enumerate(_RING_MESH):
    _MESH_TO_RING[_p] = _r
_RIGHT = [_RING_MESH[(_MESH_TO_RING[p] + 1) % N_DEV] for p in range(N_DEV)]
_LEFT = [_RING_MESH[(_MESH_TO_RING[p] - 1) % N_DEV] for p in range(N_DEV)]


def _body(scal_ref, partial_ref, out_ref, acc_ref, loc_ref,
          copy_sem, rs_send, rs_recv, ag_send, ag_recv, credit):
    my_ring = scal_ref[0]
    right = scal_ref[1]
    left = scal_ref[2]

    def prow(ref, c):
        return ref.at[pl.ds(c * MC, MC), :]

    def chunk_id(k):
        return lax.rem(my_ring - k + 2 * N_DEV, N_DEV)

    cp = pltpu.make_async_copy(
        prow(partial_ref, chunk_id(0)), acc_ref.at[0], copy_sem)
    cp.start()
    cp.wait()

    for s in range(N_DEV - 1):
        cur, nxt = s % 2, (s + 1) % 2
        if s >= 1:
            pl.semaphore_wait(credit, 1)
        rdma = pltpu.make_async_remote_copy(
            src_ref=acc_ref.at[cur],
            dst_ref=acc_ref.at[nxt],
            send_sem=rs_send.at[s],
            recv_sem=rs_recv.at[s],
            device_id=right,
            device_id_type=pl.DeviceIdType.LOGICAL,
        )
        rdma.start()
        cp_loc = pltpu.make_async_copy(
            prow(partial_ref, chunk_id(s + 1)), loc_ref, copy_sem)
        cp_loc.start()
        rdma.wait()
        if s < N_DEV - 2:
            pl.semaphore_signal(
                credit, inc=1, device_id=left,
                device_id_type=pl.DeviceIdType.LOGICAL)
        cp_loc.wait()
        acc_ref[nxt] = acc_ref[nxt] + loc_ref[...]

    fin = (N_DEV - 1) % 2
    acc_ref[fin] = jnp.maximum(acc_ref[fin], 0.0)
    c_fin = chunk_id(N_DEV - 1)
    cp_out = pltpu.make_async_copy(
        acc_ref.at[fin], prow(out_ref, c_fin), copy_sem)
    cp_out.start()
    cp_out.wait()

    for t in range(N_DEV - 1):
        g = lax.rem(my_ring + 1 - t + 2 * N_DEV, N_DEV)
        src = acc_ref.at[fin] if t == 0 else prow(out_ref, g)
        rdma = pltpu.make_async_remote_copy(
            src_ref=src,
            dst_ref=prow(out_ref, g),
            send_sem=ag_send.at[t],
            recv_sem=ag_recv.at[t],
            device_id=right,
            device_id_type=pl.DeviceIdType.LOGICAL,
        )
        rdma.start()
        rdma.wait()


def kernel(x, w_mat):
    partial = jnp.dot(x, w_mat, preferred_element_type=jnp.float32)

    p = lax.axis_index("i")
    my_ring = jnp.asarray(_MESH_TO_RING, jnp.int32)[p]
    right = jnp.asarray(_RIGHT, jnp.int32)[p]
    left = jnp.asarray(_LEFT, jnp.int32)[p]
    scal = jnp.stack([my_ring, right, left]).astype(jnp.int32)

    return pl.pallas_call(
        _body,
        out_shape=jax.ShapeDtypeStruct((M, N), jnp.float32),
        in_specs=[
            pl.BlockSpec(memory_space=pltpu.SMEM),
            pl.BlockSpec(memory_space=pl.ANY),
        ],
        out_specs=pl.BlockSpec(memory_space=pl.ANY),
        scratch_shapes=[
            pltpu.VMEM((2, MC, N), jnp.float32),
            pltpu.VMEM((MC, N), jnp.float32),
            pltpu.SemaphoreType.DMA,
            pltpu.SemaphoreType.DMA((N_DEV - 1,)),
            pltpu.SemaphoreType.DMA((N_DEV - 1,)),
            pltpu.SemaphoreType.DMA((N_DEV - 1,)),
            pltpu.SemaphoreType.DMA((N_DEV - 1,)),
            pltpu.SemaphoreType.REGULAR,
        ],
    )(scal, partial)


# device time: 1528647 ns/iter; 1.9164x vs baseline; 1.9164x over previous
import jax
import jax.numpy as jnp
from jax import lax
from jax.experimental import pallas as pl
from jax.experimental.pallas import tpu as pltpu

N_DEV = 16
M, N = 8192, 4096
MC = M // N_DEV
H = MC // 2

_PLANE = {(0, 0): 0, (1, 0): 1, (1, 1): 2, (0, 1): 3}
_RING_COORDS = (
    [(0, 0, z) for z in range(4)]
    + [(1, 0, z) for z in range(3, -1, -1)]
    + [(1, 1, z) for z in range(4)]
    + [(0, 1, z) for z in range(3, -1, -1)]
)
_RING_MESH = [4 * z + _PLANE[(x, y)] for (x, y, z) in _RING_COORDS]
_MESH_TO_RING = [0] * N_DEV
for _r, _p in enumerate(_RING_MESH):
    _MESH_TO_RING[_p] = _r
_RIGHT = [_RING_MESH[(_MESH_TO_RING[p] + 1) % N_DEV] for p in range(N_DEV)]
_LEFT = [_RING_MESH[(_MESH_TO_RING[p] - 1) % N_DEV] for p in range(N_DEV)]

_F32 = jnp.float32


def _body(scal_ref, x_ref, w_ref, out_ref, accA, accB, locA, locB,
          copy_sems, rsA_send, rsA_recv, rsB_send, rsB_recv,
          agA_send, agA_recv, agB_send, agB_recv, creditA, creditB):
    my_ring = scal_ref[0]
    right = scal_ref[1]
    left = scal_ref[2]

    def rowsA(ref, c):
        return ref.at[pl.ds(c * MC, H), :]

    def rowsB(ref, c):
        return ref.at[pl.ds(c * MC + H, H), :]

    def cw(k):
        return lax.rem(my_ring - k + 2 * N_DEV, N_DEV)

    def ccw(k):
        return lax.rem(my_ring + k, N_DEV)

    def partial(c, half_off):
        xs = x_ref[pl.ds(c * MC + half_off, H), :]
        return jnp.dot(xs, w_ref[...], preferred_element_type=_F32)

    barrier = pltpu.get_barrier_semaphore()
    for nbr in (left, right):
        pl.semaphore_signal(barrier, inc=1, device_id=nbr,
                            device_id_type=pl.DeviceIdType.LOGICAL)
    pl.semaphore_wait(barrier, 2)

    accA[0] = partial(my_ring, 0)
    accB[0] = partial(my_ring, H)

    for s in range(N_DEV - 1):
        cur, nxt = s % 2, (s + 1) % 2
        if s >= 1:
            pl.semaphore_wait(creditA, 1)
            pl.semaphore_wait(creditB, 1)
        ra = pltpu.make_async_remote_copy(
            src_ref=accA.at[cur], dst_ref=accA.at[nxt],
            send_sem=rsA_send.at[s], recv_sem=rsA_recv.at[s],
            device_id=right, device_id_type=pl.DeviceIdType.LOGICAL)
        rb = pltpu.make_async_remote_copy(
            src_ref=accB.at[cur], dst_ref=accB.at[nxt],
            send_sem=rsB_send.at[s], recv_sem=rsB_recv.at[s],
            device_id=left, device_id_type=pl.DeviceIdType.LOGICAL)
        ra.start()
        rb.start()
        locA[...] = partial(cw(s + 1), 0)
        locB[...] = partial(ccw(s + 1), H)
        ra.wait()
        rb.wait()
        if s < N_DEV - 2:
            pl.semaphore_signal(creditA, inc=1, device_id=left,
                                device_id_type=pl.DeviceIdType.LOGICAL)
            pl.semaphore_signal(creditB, inc=1, device_id=right,
                                device_id_type=pl.DeviceIdType.LOGICAL)
        accA[nxt] = accA[nxt] + locA[...]
        accB[nxt] = accB[nxt] + locB[...]

    fin = (N_DEV - 1) % 2
    accA[fin] = jnp.maximum(accA[fin], 0.0)
    accB[fin] = jnp.maximum(accB[fin], 0.0)
    cfA = cw(N_DEV - 1)
    cfB = ccw(N_DEV - 1)
    cpA = pltpu.make_async_copy(accA.at[fin], rowsA(out_ref, cfA),
                                copy_sems.at[0])
    cpB = pltpu.make_async_copy(accB.at[fin], rowsB(out_ref, cfB),
                                copy_sems.at[1])
    cpA.start()
    cpB.start()
    cpA.wait()
    cpB.wait()

    for t in range(N_DEV - 1):
        gA = lax.rem(my_ring + 1 - t + 2 * N_DEV, N_DEV)
        gB = lax.rem(my_ring - 1 + t + 2 * N_DEV, N_DEV)
        srcA = accA.at[fin] if t == 0 else rowsA(out_ref, gA)
        srcB = accB.at[fin] if t == 0 else rowsB(out_ref, gB)
        ra = pltpu.make_async_remote_copy(
            src_ref=srcA, dst_ref=rowsA(out_ref, gA),
            send_sem=agA_send.at[t], recv_sem=agA_recv.at[t],
            device_id=right, device_id_type=pl.DeviceIdType.LOGICAL)
        rb = pltpu.make_async_remote_copy(
            src_ref=srcB, dst_ref=rowsB(out_ref, gB),
            send_sem=agB_send.at[t], recv_sem=agB_recv.at[t],
            device_id=left, device_id_type=pl.DeviceIdType.LOGICAL)
        ra.start()
        rb.start()
        ra.wait()
        rb.wait()


def kernel(x, w_mat):
    p = lax.axis_index("i")
    my_ring = jnp.asarray(_MESH_TO_RING, jnp.int32)[p]
    right = jnp.asarray(_RIGHT, jnp.int32)[p]
    left = jnp.asarray(_LEFT, jnp.int32)[p]
    scal = jnp.stack([my_ring, right, left]).astype(jnp.int32)

    return pl.pallas_call(
        _body,
        out_shape=jax.ShapeDtypeStruct((M, N), jnp.float32),
        in_specs=[
            pl.BlockSpec(memory_space=pltpu.SMEM),
            pl.BlockSpec(memory_space=pltpu.VMEM),
            pl.BlockSpec(memory_space=pltpu.VMEM),
        ],
        out_specs=pl.BlockSpec(memory_space=pl.ANY),
        scratch_shapes=[
            pltpu.VMEM((2, H, N), jnp.float32),
            pltpu.VMEM((2, H, N), jnp.float32),
            pltpu.VMEM((H, N), jnp.float32),
            pltpu.VMEM((H, N), jnp.float32),
            pltpu.SemaphoreType.DMA((2,)),
            pltpu.SemaphoreType.DMA((N_DEV - 1,)),
            pltpu.SemaphoreType.DMA((N_DEV - 1,)),
            pltpu.SemaphoreType.DMA((N_DEV - 1,)),
            pltpu.SemaphoreType.DMA((N_DEV - 1,)),
            pltpu.SemaphoreType.DMA((N_DEV - 1,)),
            pltpu.SemaphoreType.DMA((N_DEV - 1,)),
            pltpu.SemaphoreType.DMA((N_DEV - 1,)),
            pltpu.SemaphoreType.DMA((N_DEV - 1,)),
            pltpu.SemaphoreType.REGULAR,
            pltpu.SemaphoreType.REGULAR,
        ],
        compiler_params=pltpu.CompilerParams(
            collective_id=0, vmem_limit_bytes=100 * 1024 * 1024),
    )(scal, x, w_mat)


# device time: 1461846 ns/iter; 2.0040x vs baseline; 1.0457x over previous
import jax
import jax.numpy as jnp
from jax import lax
from jax.experimental import pallas as pl
from jax.experimental.pallas import tpu as pltpu

N_DEV = 16
M, N = 8192, 4096
MC = M // N_DEV
H = MC // 2
Q = 2
HQ = H // Q
NSLOT = 4

_PLANE = {(0, 0): 0, (1, 0): 1, (1, 1): 2, (0, 1): 3}
_RING_COORDS = (
    [(0, 0, z) for z in range(4)]
    + [(1, 0, z) for z in range(3, -1, -1)]
    + [(1, 1, z) for z in range(4)]
    + [(0, 1, z) for z in range(3, -1, -1)]
)
_RING_MESH = [4 * z + _PLANE[(x, y)] for (x, y, z) in _RING_COORDS]
_MESH_TO_RING = [0] * N_DEV
for _r, _p in enumerate(_RING_MESH):
    _MESH_TO_RING[_p] = _r
_RIGHT = [_RING_MESH[(_MESH_TO_RING[p] + 1) % N_DEV] for p in range(N_DEV)]
_LEFT = [_RING_MESH[(_MESH_TO_RING[p] - 1) % N_DEV] for p in range(N_DEV)]

_F32 = jnp.float32
_LOGICAL = pl.DeviceIdType.LOGICAL


def _body(scal_ref, x_ref, w_ref, out_ref, accA, accB, locA, locB,
          copy_sems, rsA_send, rsA_recv, rsB_send, rsB_recv,
          agA_send, agA_recv, agB_send, agB_recv, creditA, creditB):
    my_ring = scal_ref[0]
    right = scal_ref[1]
    left = scal_ref[2]

    def cw(k):
        return lax.rem(my_ring - k + 2 * N_DEV, N_DEV)

    def ccw(k):
        return lax.rem(my_ring + k, N_DEV)

    def subA(slot, q):
        return accA.at[slot, pl.ds(q * HQ, HQ), :]

    def subB(slot, q):
        return accB.at[slot, pl.ds(q * HQ, HQ), :]

    def outA(c, q):
        return out_ref.at[pl.ds(c * MC + q * HQ, HQ), :]

    def outB(c, q):
        return out_ref.at[pl.ds(c * MC + H + q * HQ, HQ), :]

    def partial(c, half_off):
        xs = x_ref[pl.ds(c * MC + half_off, H), :]
        return jnp.dot(xs, w_ref[...], preferred_element_type=_F32)

    def rs_send(ring_sub, slot_src, slot_dst, sems_s, sems_r, s, q, dev):
        d = pltpu.make_async_remote_copy(
            src_ref=ring_sub(slot_src, q), dst_ref=ring_sub(slot_dst, q),
            send_sem=sems_s.at[s, q], recv_sem=sems_r.at[s, q],
            device_id=dev, device_id_type=_LOGICAL)
        d.start()
        return d

    barrier = pltpu.get_barrier_semaphore()
    for nbr in (left, right):
        pl.semaphore_signal(barrier, inc=1, device_id=nbr,
                            device_id_type=_LOGICAL)
    pl.semaphore_wait(barrier, 2)

    accA[0] = partial(my_ring, 0)
    accB[0] = partial(my_ring, H)
    prevA = [rs_send(subA, 0, 1, rsA_send, rsA_recv, 0, q, right)
             for q in range(Q)]
    prevB = [rs_send(subB, 0, 1, rsB_send, rsB_recv, 0, q, left)
             for q in range(Q)]

    for s in range(N_DEV - 1):
        rv = (s + 1) % NSLOT
        locA[...] = partial(cw(s + 1), 0)
        locB[...] = partial(ccw(s + 1), H)
        newA, newB = [], []
        for q in range(Q):
            rA = pltpu.make_async_remote_copy(
                src_ref=subA(rv, q), dst_ref=subA(rv, q),
                send_sem=rsA_send.at[s, q], recv_sem=rsA_recv.at[s, q],
                device_id=left, device_id_type=_LOGICAL)
            rA.wait_recv()
            accA[rv, pl.ds(q * HQ, HQ), :] = (
                accA[rv, pl.ds(q * HQ, HQ), :] + locA[pl.ds(q * HQ, HQ), :])
            if s < N_DEV - 2:
                if s + 1 >= 3:
                    pl.semaphore_wait(creditA, 1)
                newA.append(rs_send(subA, rv, (s + 2) % NSLOT,
                                    rsA_send, rsA_recv, s + 1, q, right))
            rB = pltpu.make_async_remote_copy(
                src_ref=subB(rv, q), dst_ref=subB(rv, q),
                send_sem=rsB_send.at[s, q], recv_sem=rsB_recv.at[s, q],
                device_id=right, device_id_type=_LOGICAL)
            rB.wait_recv()
            accB[rv, pl.ds(q * HQ, HQ), :] = (
                accB[rv, pl.ds(q * HQ, HQ), :] + locB[pl.ds(q * HQ, HQ), :])
            if s < N_DEV - 2:
                if s + 1 >= 3:
                    pl.semaphore_wait(creditB, 1)
                newB.append(rs_send(subB, rv, (s + 2) % NSLOT,
                                    rsB_send, rsB_recv, s + 1, q, left))
        for d in prevA + prevB:
            d.wait_send()
        if s + 3 <= N_DEV - 2:
            for q in range(Q):
                pl.semaphore_signal(creditA, inc=1, device_id=left,
                                    device_id_type=_LOGICAL)
                pl.semaphore_signal(creditB, inc=1, device_id=right,
                                    device_id_type=_LOGICAL)
        prevA, prevB = newA, newB

    fin = (N_DEV - 1) % NSLOT
    accA[fin] = jnp.maximum(accA[fin], 0.0)
    accB[fin] = jnp.maximum(accB[fin], 0.0)
    cfA = cw(N_DEV - 1)
    cfB = ccw(N_DEV - 1)

    ag_pend = []
    for q in range(Q):
        d = pltpu.make_async_remote_copy(
            src_ref=subA(fin, q), dst_ref=outA(cfA, q),
            send_sem=agA_send.at[0, q], recv_sem=agA_recv.at[0, q],
            device_id=right, device_id_type=_LOGICAL)
        d.start()
        ag_pend.append(d)
        d = pltpu.make_async_remote_copy(
            src_ref=subB(fin, q), dst_ref=outB(cfB, q),
            send_sem=agB_send.at[0, q], recv_sem=agB_recv.at[0, q],
            device_id=left, device_id_type=_LOGICAL)
        d.start()
        ag_pend.append(d)

    cp_own = []
    for q in range(Q):
        c = pltpu.make_async_copy(subA(fin, q), outA(cfA, q),
                                  copy_sems.at[0, q])
        c.start()
        cp_own.append(c)
        c = pltpu.make_async_copy(subB(fin, q), outB(cfB, q),
                                  copy_sems.at[1, q])
        c.start()
        cp_own.append(c)

    for t in range(N_DEV - 1):
        rAc = lax.rem(my_ring - t + 2 * N_DEV, N_DEV)
        rBc = lax.rem(my_ring + t, N_DEV)
        for q in range(Q):
            rA = pltpu.make_async_remote_copy(
                src_ref=outA(rAc, q), dst_ref=outA(rAc, q),
                send_sem=agA_send.at[t, q], recv_sem=agA_recv.at[t, q],
                device_id=left, device_id_type=_LOGICAL)
            rA.wait_recv()
            if t < N_DEV - 2:
                d = pltpu.make_async_remote_copy(
                    src_ref=outA(rAc, q), dst_ref=outA(rAc, q),
                    send_sem=agA_send.at[t + 1, q],
                    recv_sem=agA_recv.at[t + 1, q],
                    device_id=right, device_id_type=_LOGICAL)
                d.start()
                ag_pend.append(d)
            rB = pltpu.make_async_remote_copy(
                src_ref=outB(rBc, q), dst_ref=outB(rBc, q),
                send_sem=agB_send.at[t, q], recv_sem=agB_recv.at[t, q],
                device_id=right, device_id_type=_LOGICAL)
            rB.wait_recv()
            if t < N_DEV - 2:
                d = pltpu.make_async_remote_copy(
                    src_ref=outB(rBc, q), dst_ref=outB(rBc, q),
                    send_sem=agB_send.at[t + 1, q],
                    recv_sem=agB_recv.at[t + 1, q],
                    device_id=left, device_id_type=_LOGICAL)
                d.start()
                ag_pend.append(d)

    for c in cp_own:
        c.wait()
    for d in ag_pend:
        d.wait_send()


def kernel(x, w_mat):
    p = lax.axis_index("i")
    my_ring = jnp.asarray(_MESH_TO_RING, jnp.int32)[p]
    right = jnp.asarray(_RIGHT, jnp.int32)[p]
    left = jnp.asarray(_LEFT, jnp.int32)[p]
    scal = jnp.stack([my_ring, right, left]).astype(jnp.int32)

    return pl.pallas_call(
        _body,
        out_shape=jax.ShapeDtypeStruct((M, N), jnp.float32),
        in_specs=[
            pl.BlockSpec(memory_space=pltpu.SMEM),
            pl.BlockSpec(memory_space=pltpu.VMEM),
            pl.BlockSpec(memory_space=pltpu.VMEM),
        ],
        out_specs=pl.BlockSpec(memory_space=pl.ANY),
        scratch_shapes=[
            pltpu.VMEM((NSLOT, H, N), jnp.float32),
            pltpu.VMEM((NSLOT, H, N), jnp.float32),
            pltpu.VMEM((H, N), jnp.float32),
            pltpu.VMEM((H, N), jnp.float32),
            pltpu.SemaphoreType.DMA((2, Q)),
            pltpu.SemaphoreType.DMA((N_DEV - 1, Q)),
            pltpu.SemaphoreType.DMA((N_DEV - 1, Q)),
            pltpu.SemaphoreType.DMA((N_DEV - 1, Q)),
            pltpu.SemaphoreType.DMA((N_DEV - 1, Q)),
            pltpu.SemaphoreType.DMA((N_DEV - 1, Q)),
            pltpu.SemaphoreType.DMA((N_DEV - 1, Q)),
            pltpu.SemaphoreType.DMA((N_DEV - 1, Q)),
            pltpu.SemaphoreType.DMA((N_DEV - 1, Q)),
            pltpu.SemaphoreType.REGULAR,
            pltpu.SemaphoreType.REGULAR,
        ],
        compiler_params=pltpu.CompilerParams(
            collective_id=0, vmem_limit_bytes=100 * 1024 * 1024),
    )(scal, x, w_mat)


# device time: 1459170 ns/iter; 2.0077x vs baseline; 1.0018x over previous
import jax
import jax.numpy as jnp
from jax import lax
from jax.experimental import pallas as pl
from jax.experimental.pallas import tpu as pltpu

N_DEV = 16
M, N = 8192, 4096
MC = M // N_DEV
H = MC // 2
Q = 2
HQ = H // Q
NSLOT = 4

_PLANE = {(0, 0): 0, (1, 0): 1, (1, 1): 2, (0, 1): 3}
_RING_COORDS = (
    [(0, 0, z) for z in range(4)]
    + [(1, 0, z) for z in range(3, -1, -1)]
    + [(1, 1, z) for z in range(4)]
    + [(0, 1, z) for z in range(3, -1, -1)]
)
_RING_MESH = [4 * z + _PLANE[(x, y)] for (x, y, z) in _RING_COORDS]
_MESH_TO_RING = [0] * N_DEV
for _r, _p in enumerate(_RING_MESH):
    _MESH_TO_RING[_p] = _r
_RIGHT = [_RING_MESH[(_MESH_TO_RING[p] + 1) % N_DEV] for p in range(N_DEV)]
_LEFT = [_RING_MESH[(_MESH_TO_RING[p] - 1) % N_DEV] for p in range(N_DEV)]

_F32 = jnp.float32
_LOGICAL = pl.DeviceIdType.LOGICAL


def _body(scal_ref, x_ref, w_ref, out_ref, accA, accB, locA, locB,
          copy_sems, rsA_send, rsA_recv, rsB_send, rsB_recv,
          agA_send, agA_recv, agB_send, agB_recv, creditA, creditB):
    my_ring = scal_ref[0]
    right = scal_ref[1]
    left = scal_ref[2]

    def cw(k):
        return lax.rem(my_ring - k + 2 * N_DEV, N_DEV)

    def ccw(k):
        return lax.rem(my_ring + k, N_DEV)

    def subA(slot, q):
        return accA.at[slot, pl.ds(q * HQ, HQ), :]

    def subB(slot, q):
        return accB.at[slot, pl.ds(q * HQ, HQ), :]

    def outA(c, q):
        return out_ref.at[pl.ds(c * MC + q * HQ, HQ), :]

    def outB(c, q):
        return out_ref.at[pl.ds(c * MC + H + q * HQ, HQ), :]

    def partial(c, half_off):
        xs = x_ref[pl.ds(c * MC + half_off, H), :]
        return jnp.dot(xs, w_ref[...], preferred_element_type=_F32)

    def rs_send(ring_sub, slot_src, slot_dst, sems_s, sems_r, s, q, dev):
        d = pltpu.make_async_remote_copy(
            src_ref=ring_sub(slot_src, q), dst_ref=ring_sub(slot_dst, q),
            send_sem=sems_s.at[s, q], recv_sem=sems_r.at[s, q],
            device_id=dev, device_id_type=_LOGICAL)
        d.start()
        return d

    barrier = pltpu.get_barrier_semaphore()
    for nbr in (left, right):
        pl.semaphore_signal(barrier, inc=1, device_id=nbr,
                            device_id_type=_LOGICAL)
    accA[0] = partial(my_ring, 0)
    accB[0] = partial(my_ring, H)
    pl.semaphore_wait(barrier, 2)
    prevA = [rs_send(subA, 0, 1, rsA_send, rsA_recv, 0, q, right)
             for q in range(Q)]
    prevB = [rs_send(subB, 0, 1, rsB_send, rsB_recv, 0, q, left)
             for q in range(Q)]

    fin = (N_DEV - 1) % NSLOT
    cfA = cw(N_DEV - 1)
    cfB = ccw(N_DEV - 1)
    ag_pend = []
    last = N_DEV - 2
    for s in range(N_DEV - 1):
        rv = (s + 1) % NSLOT
        locA[...] = partial(cw(s + 1), 0)
        locB[...] = partial(ccw(s + 1), H)
        newA, newB = [], []
        for q in range(Q):
            sl = pl.ds(q * HQ, HQ)
            rA = pltpu.make_async_remote_copy(
                src_ref=subA(rv, q), dst_ref=subA(rv, q),
                send_sem=rsA_send.at[s, q], recv_sem=rsA_recv.at[s, q],
                device_id=left, device_id_type=_LOGICAL)
            rA.wait_recv()
            sumA = accA[rv, sl, :] + locA[sl, :]
            if s == last:
                sumA = jnp.maximum(sumA, 0.0)
            accA[rv, sl, :] = sumA
            if s < last:
                if s + 1 >= 3:
                    pl.semaphore_wait(creditA, 1)
                newA.append(rs_send(subA, rv, (s + 2) % NSLOT,
                                    rsA_send, rsA_recv, s + 1, q, right))
            else:
                d = pltpu.make_async_remote_copy(
                    src_ref=subA(rv, q), dst_ref=outA(cfA, q),
                    send_sem=agA_send.at[0, q], recv_sem=agA_recv.at[0, q],
                    device_id=right, device_id_type=_LOGICAL)
                d.start()
                ag_pend.append(d)
            rB = pltpu.make_async_remote_copy(
                src_ref=subB(rv, q), dst_ref=subB(rv, q),
                send_sem=rsB_send.at[s, q], recv_sem=rsB_recv.at[s, q],
                device_id=right, device_id_type=_LOGICAL)
            rB.wait_recv()
            sumB = accB[rv, sl, :] + locB[sl, :]
            if s == last:
                sumB = jnp.maximum(sumB, 0.0)
            accB[rv, sl, :] = sumB
            if s < last:
                if s + 1 >= 3:
                    pl.semaphore_wait(creditB, 1)
                newB.append(rs_send(subB, rv, (s + 2) % NSLOT,
                                    rsB_send, rsB_recv, s + 1, q, left))
            else:
                d = pltpu.make_async_remote_copy(
                    src_ref=subB(rv, q), dst_ref=outB(cfB, q),
                    send_sem=agB_send.at[0, q], recv_sem=agB_recv.at[0, q],
                    device_id=left, device_id_type=_LOGICAL)
                d.start()
                ag_pend.append(d)
        for d in prevA + prevB:
            d.wait_send()
        if s + 3 <= N_DEV - 2:
            for q in range(Q):
                pl.semaphore_signal(creditA, inc=1, device_id=left,
                                    device_id_type=_LOGICAL)
                pl.semaphore_signal(creditB, inc=1, device_id=right,
                                    device_id_type=_LOGICAL)
        prevA, prevB = newA, newB


    cp_own = []
    for q in range(Q):
        c = pltpu.make_async_copy(subA(fin, q), outA(cfA, q),
                                  copy_sems.at[0, q])
        c.start()
        cp_own.append(c)
        c = pltpu.make_async_copy(subB(fin, q), outB(cfB, q),
                                  copy_sems.at[1, q])
        c.start()
        cp_own.append(c)

    for t in range(N_DEV - 1):
        rAc = lax.rem(my_ring - t + 2 * N_DEV, N_DEV)
        rBc = lax.rem(my_ring + t, N_DEV)
        for q in range(Q):
            rA = pltpu.make_async_remote_copy(
                src_ref=outA(rAc, q), dst_ref=outA(rAc, q),
                send_sem=agA_send.at[t, q], recv_sem=agA_recv.at[t, q],
                device_id=left, device_id_type=_LOGICAL)
            rA.wait_recv()
            if t < N_DEV - 2:
                d = pltpu.make_async_remote_copy(
                    src_ref=outA(rAc, q), dst_ref=outA(rAc, q),
                    send_sem=agA_send.at[t + 1, q],
                    recv_sem=agA_recv.at[t + 1, q],
                    device_id=right, device_id_type=_LOGICAL)
                d.start()
                ag_pend.append(d)
            rB = pltpu.make_async_remote_copy(
                src_ref=outB(rBc, q), dst_ref=outB(rBc, q),
                send_sem=agB_send.at[t, q], recv_sem=agB_recv.at[t, q],
                device_id=right, device_id_type=_LOGICAL)
            rB.wait_recv()
            if t < N_DEV - 2:
                d = pltpu.make_async_remote_copy(
                    src_ref=outB(rBc, q), dst_ref=outB(rBc, q),
                    send_sem=agB_send.at[t + 1, q],
                    recv_sem=agB_recv.at[t + 1, q],
                    device_id=left, device_id_type=_LOGICAL)
                d.start()
                ag_pend.append(d)

    for c in cp_own:
        c.wait()
    for d in ag_pend:
        d.wait_send()


def kernel(x, w_mat):
    p = lax.axis_index("i")
    my_ring = jnp.asarray(_MESH_TO_RING, jnp.int32)[p]
    right = jnp.asarray(_RIGHT, jnp.int32)[p]
    left = jnp.asarray(_LEFT, jnp.int32)[p]
    scal = jnp.stack([my_ring, right, left]).astype(jnp.int32)

    return pl.pallas_call(
        _body,
        out_shape=jax.ShapeDtypeStruct((M, N), jnp.float32),
        in_specs=[
            pl.BlockSpec(memory_space=pltpu.SMEM),
            pl.BlockSpec(memory_space=pltpu.VMEM),
            pl.BlockSpec(memory_space=pltpu.VMEM),
        ],
        out_specs=pl.BlockSpec(memory_space=pl.ANY),
        scratch_shapes=[
            pltpu.VMEM((NSLOT, H, N), jnp.float32),
            pltpu.VMEM((NSLOT, H, N), jnp.float32),
            pltpu.VMEM((H, N), jnp.float32),
            pltpu.VMEM((H, N), jnp.float32),
            pltpu.SemaphoreType.DMA((2, Q)),
            pltpu.SemaphoreType.DMA((N_DEV - 1, Q)),
            pltpu.SemaphoreType.DMA((N_DEV - 1, Q)),
            pltpu.SemaphoreType.DMA((N_DEV - 1, Q)),
            pltpu.SemaphoreType.DMA((N_DEV - 1, Q)),
            pltpu.SemaphoreType.DMA((N_DEV - 1, Q)),
            pltpu.SemaphoreType.DMA((N_DEV - 1, Q)),
            pltpu.SemaphoreType.DMA((N_DEV - 1, Q)),
            pltpu.SemaphoreType.DMA((N_DEV - 1, Q)),
            pltpu.SemaphoreType.REGULAR,
            pltpu.SemaphoreType.REGULAR,
        ],
        compiler_params=pltpu.CompilerParams(
            collective_id=0, vmem_limit_bytes=100 * 1024 * 1024),
    )(scal, x, w_mat)
